# Initial kernel scaffold; baseline (speedup 1.0000x reference)
import functools

import jax
import jax.numpy as jnp
from jax import lax
from jax.experimental import pallas as pl
from jax.experimental.pallas import tpu as pltpu
from jax.experimental.pallas import tpu_sc as plsc

NC = 2
NS = 16
LANES = 16


def _node_linear_body(x_ref, ws_ref, bs_ref, wd_ref, bd_ref, wdu_ref, bdu_ref,
                      wsu_ref, bsu_ref, srctab_ref, edtab_ref, xsu_ref):
    H = x_ref.shape[1] // 2
    xb = x_ref[...]
    dn = (((1,), (1,)), ((), ()))
    es = lax.dot_general(xb, ws_ref[...], dn,
                         preferred_element_type=jnp.float32) + bs_ref[...]
    ed = lax.dot_general(xb, wd_ref[...], dn,
                         preferred_element_type=jnp.float32) + bd_ref[...]
    bh = lax.dot_general(xb, wdu_ref[...], dn,
                         preferred_element_type=jnp.float32) + bdu_ref[...]
    xsu_ref[...] = lax.dot_general(xb, wsu_ref[...], dn,
                                   preferred_element_type=jnp.float32) + bsu_ref[...]
    srctab_ref[0] = jnp.concatenate([es[:, :H], bh[:, :H]], axis=1)
    srctab_ref[1] = jnp.concatenate([es[:, H:], bh[:, H:]], axis=1)
    edtab_ref[0] = ed[:, :H]
    edtab_ref[1] = ed[:, H:]


def _node_linear(x, W_src, b_src, W_dst, b_dst, W_du, b_du, W_su, b_su):
    N, D = x.shape
    H = D // 2
    BN = 2000
    wspec = pl.BlockSpec((D, D), lambda i: (0, 0))
    bspec = pl.BlockSpec((1, D), lambda i: (0, 0))
    return pl.pallas_call(
        _node_linear_body,
        grid=(N // BN,),
        in_specs=[
            pl.BlockSpec((BN, D), lambda i: (i, 0)),
            wspec, bspec, wspec, bspec, wspec, bspec, wspec, bspec,
        ],
        out_specs=[
            pl.BlockSpec((2, BN, D), lambda i: (0, i, 0)),
            pl.BlockSpec((2, BN, H), lambda i: (0, i, 0)),
            pl.BlockSpec((BN, D), lambda i: (i, 0)),
        ],
        out_shape=[
            jax.ShapeDtypeStruct((2, N, D), jnp.float32),
            jax.ShapeDtypeStruct((2, N, H), jnp.float32),
            jax.ShapeDtypeStruct((N, D), jnp.float32),
        ],
    )(x, W_src, b_src.reshape(1, D), W_dst, b_dst.reshape(1, D),
      W_du, b_du.reshape(1, D), W_su, b_su.reshape(1, D))


def _edge_linear_body(ea_ref, w_ref, b_ref, out_ref):
    H = ea_ref.shape[1] // 2
    ew = lax.dot_general(ea_ref[...], w_ref[...], (((1,), (1,)), ((), ())),
                         preferred_element_type=jnp.float32) + b_ref[...]
    out_ref[0] = ew[:, :H]
    out_ref[1] = ew[:, H:]


def _edge_linear(edge_attr, W_eg, b_eg):
    E, D = edge_attr.shape
    H = D // 2
    BE = 8000
    return pl.pallas_call(
        _edge_linear_body,
        grid=(E // BE,),
        in_specs=[
            pl.BlockSpec((BE, D), lambda i: (i, 0)),
            pl.BlockSpec((D, D), lambda i: (0, 0)),
            pl.BlockSpec((1, D), lambda i: (0, 0)),
        ],
        out_specs=pl.BlockSpec((2, BE, H), lambda i: (0, i, 0)),
        out_shape=jax.ShapeDtypeStruct((2, E, H), jnp.float32),
    )(edge_attr, W_eg, b_eg.reshape(1, D))


def _sc_message(src, dst, srctab, edtab, ew, N, E, D):
    H = D // 2
    C = 80
    EPW = E // NS
    ZR = 125
    RPT = N // NS

    mesh = plsc.VectorSubcoreMesh(core_axis_name="c", subcore_axis_name="s",
                                  num_cores=NC, num_subcores=NS)

    def body(src_hbm, dst_hbm, srctab_hbm, edtab_hbm, ew_hbm,
             m_hbm, acc_hbm, stats_hbm,
             idx_src_v, idx_dst_v, idx_srcN_v, idx_dstN_v,
             srcrows_v, edrows_v, ew_v, m_v, out_v, stats_v, zero_v,
             acc_sp, sem1, sem2, sem3):
        c = lax.axis_index("c")
        s = lax.axis_index("s")
        zv = jnp.zeros((LANES,), jnp.float32)

        def zrow(r, carry):
            for j in range(D // LANES):
                zero_v[r, pl.ds(j * LANES, LANES)] = zv
            return carry
        lax.fori_loop(0, ZR, zrow, 0)
        for j in range(D // LANES):
            stats_v[pl.ds(j * LANES, LANES)] = zv

        for k in range(RPT // ZR):
            pltpu.sync_copy(zero_v, acc_sp.at[pl.ds(s * RPT + k * ZR, ZR)])
        plsc.subcore_barrier()

        base0 = s * EPW
        cN = c * N
        cE = c * E

        def chunk(g, carry):
            base = base0 + g * C
            pltpu.sync_copy(src_hbm.at[pl.ds(base, C)], idx_src_v)
            pltpu.sync_copy(dst_hbm.at[pl.ds(base, C)], idx_dst_v)
            for j in range(C // LANES):
                sl = pl.ds(j * LANES, LANES)
                idx_srcN_v[sl] = idx_src_v[sl] + cN
                idx_dstN_v[sl] = idx_dst_v[sl] + cN
            cp1 = pltpu.async_copy(srctab_hbm.at[idx_srcN_v], srcrows_v, sem1)
            cp2 = pltpu.async_copy(edtab_hbm.at[idx_dstN_v], edrows_v, sem2)
            cp3 = pltpu.async_copy(ew_hbm.at[pl.ds(cE + base, C)], ew_v, sem3)
            cp1.wait()
            cp2.wait()
            cp3.wait()

            def row(i, rcarry):
                for j in range(H // LANES):
                    sl = pl.ds(j * LANES, LANES)
                    slh = pl.ds(H + j * LANES, LANES)
                    m = srcrows_v[i, sl] + edrows_v[i, sl] + ew_v[i, sl]
                    m_v[i, sl] = m
                    sg = 1.0 / (1.0 + jnp.exp(-m))
                    out_v[i, sl] = sg * srcrows_v[i, slh]
                    out_v[i, slh] = sg
                    plsc.addupdate(stats_v.at[sl], m)
                    plsc.addupdate(stats_v.at[slh], m * m)
                return rcarry
            lax.fori_loop(0, C, row, 0)

            pltpu.sync_copy(out_v, acc_sp.at[idx_dst_v], add=True)
            pltpu.sync_copy(m_v, m_hbm.at[pl.ds(cE + base, C)])
            return carry
        lax.fori_loop(0, EPW // C, chunk, 0)

        pltpu.sync_copy(stats_v, stats_hbm.at[c * NS + s])
        plsc.subcore_barrier()
        for k in range(RPT // ZR):
            r0 = s * RPT + k * ZR
            pltpu.sync_copy(acc_sp.at[pl.ds(r0, ZR)],
                            acc_hbm.at[pl.ds(cN + r0, ZR)])

    run = pl.kernel(
        body,
        out_type=[
            jax.ShapeDtypeStruct((2 * E, H), jnp.float32),
            jax.ShapeDtypeStruct((2 * N, D), jnp.float32),
            jax.ShapeDtypeStruct((NC * NS, D), jnp.float32),
        ],
        mesh=mesh,
        scratch_types=[
            pltpu.VMEM((C,), jnp.int32),
            pltpu.VMEM((C,), jnp.int32),
            pltpu.VMEM((C,), jnp.int32),
            pltpu.VMEM((C,), jnp.int32),
            pltpu.VMEM((C, D), jnp.float32),
            pltpu.VMEM((C, H), jnp.float32),
            pltpu.VMEM((C, H), jnp.float32),
            pltpu.VMEM((C, H), jnp.float32),
            pltpu.VMEM((C, D), jnp.float32),
            pltpu.VMEM((D,), jnp.float32),
            pltpu.VMEM((ZR, D), jnp.float32),
            pltpu.VMEM_SHARED((N, D), jnp.float32),
            pltpu.SemaphoreType.DMA,
            pltpu.SemaphoreType.DMA,
            pltpu.SemaphoreType.DMA,
        ],
    )
    return run(src, dst, srctab, edtab, ew)


def _edge_final_body(m_ref, ea_ref, st_ref, g_ref, b_ref, y_ref, *, E):
    H = m_ref.shape[2]
    s_lo = jnp.sum(st_ref[:NS], axis=0, keepdims=True)
    s_hi = jnp.sum(st_ref[NS:], axis=0, keepdims=True)
    mean = jnp.concatenate([s_lo[:, :H], s_hi[:, :H]], axis=1) / E
    em2 = jnp.concatenate([s_lo[:, H:], s_hi[:, H:]], axis=1) / E
    var = em2 - mean * mean
    inv = lax.rsqrt(var + 1e-5)
    m = jnp.concatenate([m_ref[0], m_ref[1]], axis=1)
    yn = g_ref[...] * (m - mean) * inv + b_ref[...]
    y_ref[...] = ea_ref[...] + yn * jax.nn.sigmoid(yn)


def _edge_final(edge_attr, m_split, stats, gamma_e, beta_e):
    E, D = edge_attr.shape
    H = D // 2
    BE = 8000
    return pl.pallas_call(
        functools.partial(_edge_final_body, E=E),
        grid=(E // BE,),
        in_specs=[
            pl.BlockSpec((2, BE, H), lambda i: (0, i, 0)),
            pl.BlockSpec((BE, D), lambda i: (i, 0)),
            pl.BlockSpec((NC * NS, D), lambda i: (0, 0)),
            pl.BlockSpec((1, D), lambda i: (0, 0)),
            pl.BlockSpec((1, D), lambda i: (0, 0)),
        ],
        out_specs=pl.BlockSpec((BE, D), lambda i: (i, 0)),
        out_shape=jax.ShapeDtypeStruct((E, D), jnp.float32),
    )(m_split, edge_attr, stats, gamma_e.reshape(1, D), beta_e.reshape(1, D))


def _node_final_body(x_ref, xsu_ref, acc_ref, g_ref, b_ref, out_ref):
    D = x_ref.shape[1]
    H = D // 2
    a0 = acc_ref[0]
    a1 = acc_ref[1]
    h = jnp.concatenate([a0[:, :H] / (a0[:, H:] + 1e-6),
                         a1[:, :H] / (a1[:, H:] + 1e-6)], axis=1)
    xo = xsu_ref[...] + h
    mu = jnp.mean(xo, axis=0, keepdims=True)
    var = jnp.mean((xo - mu) * (xo - mu), axis=0, keepdims=True)
    xn = g_ref[...] * (xo - mu) * lax.rsqrt(var + 1e-5) + b_ref[...]
    out_ref[...] = x_ref[...] + xn * jax.nn.sigmoid(xn)


def _node_final(x, xsu, acc, gamma_n, beta_n):
    N, D = x.shape
    return pl.pallas_call(
        _node_final_body,
        out_shape=jax.ShapeDtypeStruct((N, D), jnp.float32),
    )(x, xsu, acc, gamma_n.reshape(1, D), beta_n.reshape(1, D))


def kernel(x, edge_index, edge_attr, W_src, b_src, W_dst, b_dst, W_eg, b_eg,
           W_su, b_su, W_du, b_du, gamma_n, beta_n, gamma_e, beta_e):
    N, D = x.shape
    E = edge_index.shape[1]
    H = D // 2
    src = edge_index[0]
    dst = edge_index[1]

    srctab, edtab, xsu = _node_linear(x, W_src, b_src, W_dst, b_dst,
                                      W_du, b_du, W_su, b_su)
    ew = _edge_linear(edge_attr, W_eg, b_eg)

    m_flat, acc_flat, stats = _sc_message(
        src, dst, srctab.reshape(2 * N, D), edtab.reshape(2 * N, H),
        ew.reshape(2 * E, H), N, E, D)

    x_out = _node_final(x, xsu, acc_flat.reshape(2, N, D), gamma_n, beta_n)
    y_out = _edge_final(edge_attr, m_flat.reshape(2, E, H), stats,
                        gamma_e, beta_e)
    return (x_out, y_out)


# speedup vs baseline: 1.3484x; 1.3484x over previous
import functools

import jax
import jax.numpy as jnp
from jax import lax
from jax.experimental import pallas as pl
from jax.experimental.pallas import tpu as pltpu
from jax.experimental.pallas import tpu_sc as plsc

NC = 2
NS = 16
LANES = 16


def _node_linear_body(x_ref, ws_ref, bs_ref, wd_ref, bd_ref, wdu_ref, bdu_ref,
                      wsu_ref, bsu_ref, srctab_ref, edtab_ref, xsu_ref):
    H = x_ref.shape[1] // 2
    xb = x_ref[...]
    dn = (((1,), (1,)), ((), ()))
    es = lax.dot_general(xb, ws_ref[...], dn,
                         preferred_element_type=jnp.float32) + bs_ref[...]
    ed = lax.dot_general(xb, wd_ref[...], dn,
                         preferred_element_type=jnp.float32) + bd_ref[...]
    bh = lax.dot_general(xb, wdu_ref[...], dn,
                         preferred_element_type=jnp.float32) + bdu_ref[...]
    xsu_ref[...] = lax.dot_general(xb, wsu_ref[...], dn,
                                   preferred_element_type=jnp.float32) + bsu_ref[...]
    srctab_ref[0] = jnp.concatenate([es[:, :H], bh[:, :H]], axis=1)
    srctab_ref[1] = jnp.concatenate([es[:, H:], bh[:, H:]], axis=1)
    edtab_ref[...] = ed


def _node_linear(x, W_src, b_src, W_dst, b_dst, W_du, b_du, W_su, b_su):
    N, D = x.shape
    H = D // 2
    BN = 2000
    wspec = pl.BlockSpec((D, D), lambda i: (0, 0))
    bspec = pl.BlockSpec((1, D), lambda i: (0, 0))
    return pl.pallas_call(
        _node_linear_body,
        grid=(N // BN,),
        in_specs=[
            pl.BlockSpec((BN, D), lambda i: (i, 0)),
            wspec, bspec, wspec, bspec, wspec, bspec, wspec, bspec,
        ],
        out_specs=[
            pl.BlockSpec((2, BN, D), lambda i: (0, i, 0)),
            pl.BlockSpec((BN, D), lambda i: (i, 0)),
            pl.BlockSpec((BN, D), lambda i: (i, 0)),
        ],
        out_shape=[
            jax.ShapeDtypeStruct((2, N, D), jnp.float32),
            jax.ShapeDtypeStruct((N, D), jnp.float32),
            jax.ShapeDtypeStruct((N, D), jnp.float32),
        ],
    )(x, W_src, b_src.reshape(1, D), W_dst, b_dst.reshape(1, D),
      W_du, b_du.reshape(1, D), W_su, b_su.reshape(1, D))


def _edge_linear_body(ea_ref, w_ref, b_ref, out_ref):
    H = ea_ref.shape[1] // 2
    ew = lax.dot_general(ea_ref[...], w_ref[...], (((1,), (1,)), ((), ())),
                         preferred_element_type=jnp.float32) + b_ref[...]
    out_ref[0] = ew[:, :H]
    out_ref[1] = ew[:, H:]


def _edge_linear(edge_attr, W_eg, b_eg):
    E, D = edge_attr.shape
    H = D // 2
    BE = 8000
    return pl.pallas_call(
        _edge_linear_body,
        grid=(E // BE,),
        in_specs=[
            pl.BlockSpec((BE, D), lambda i: (i, 0)),
            pl.BlockSpec((D, D), lambda i: (0, 0)),
            pl.BlockSpec((1, D), lambda i: (0, 0)),
        ],
        out_specs=pl.BlockSpec((2, BE, H), lambda i: (0, i, 0)),
        out_shape=jax.ShapeDtypeStruct((2, E, H), jnp.float32),
    )(edge_attr, W_eg, b_eg.reshape(1, D))


def _sc_message(src, dst, srctab, edtab, ew, N, E, D):
    H = D // 2
    C = 80
    EPW = E // NS
    ZR = 8
    NZC = N // ZR
    ZPT = (NZC + NS - 1) // NS

    mesh = plsc.VectorSubcoreMesh(core_axis_name="c", subcore_axis_name="s",
                                  num_cores=NC, num_subcores=NS)

    def body(src_hbm, dst_hbm, srctab_hbm, edtab_hbm, ew_hbm,
             m_hbm, acc_hbm, stats_hbm,
             idx_src_v, idx_dst_v, idx_srcN_v,
             srcrows_v, edrows_v, ew_v, out_v, stats_v, zero_v,
             acc_sp, sem1, sem2, sem3):
        c = lax.axis_index("c")
        s = lax.axis_index("s")
        zv = jnp.zeros((LANES,), jnp.float32)

        def zrow(r, carry):
            for j in range(D // LANES):
                zero_v[r, pl.ds(j * LANES, LANES)] = zv
            return carry
        lax.fori_loop(0, ZR, zrow, 0)
        for r in range(8):
            for j in range(D // LANES):
                stats_v[r, pl.ds(j * LANES, LANES)] = zv

        for k in range(ZPT):
            cid = k * NS + s

            @pl.when(cid < NZC)
            def _():
                pltpu.sync_copy(zero_v, acc_sp.at[pl.ds(cid * ZR, ZR)])
        plsc.subcore_barrier()

        base0 = s * EPW
        cN = c * N
        cE = c * E
        cH = c * H

        def chunk(g, carry):
            base = base0 + g * C
            pltpu.sync_copy(src_hbm.at[pl.ds(base, C)], idx_src_v)
            pltpu.sync_copy(dst_hbm.at[pl.ds(base, C)], idx_dst_v)
            for j in range(C // LANES):
                sl = pl.ds(j * LANES, LANES)
                idx_srcN_v[sl] = idx_src_v[sl] + cN
            cp1 = pltpu.async_copy(srctab_hbm.at[idx_srcN_v], srcrows_v, sem1)
            cp2 = pltpu.async_copy(edtab_hbm.at[idx_dst_v], edrows_v, sem2)
            cp3 = pltpu.async_copy(ew_hbm.at[pl.ds(cE + base, C)], ew_v, sem3)
            cp1.wait()
            cp2.wait()
            cp3.wait()

            def row(i, rcarry):
                for j in range(H // LANES):
                    sl = pl.ds(j * LANES, LANES)
                    slh = pl.ds(H + j * LANES, LANES)
                    m = (srcrows_v[i, sl]
                         + edrows_v[i, pl.ds(cH + j * LANES, LANES)]
                         + ew_v[i, sl])
                    ew_v[i, sl] = m
                    sg = 1.0 / (1.0 + jnp.exp(-m))
                    out_v[i, sl] = sg * srcrows_v[i, slh]
                    out_v[i, slh] = sg
                    plsc.addupdate(stats_v.at[0, sl], m)
                    plsc.addupdate(stats_v.at[0, slh], m * m)
                return rcarry
            lax.fori_loop(0, C, row, 0)

            pltpu.sync_copy(out_v, acc_sp.at[idx_dst_v], add=True)
            pltpu.sync_copy(ew_v, m_hbm.at[pl.ds(cE + base, C)])
            return carry
        lax.fori_loop(0, EPW // C, chunk, 0)

        pltpu.sync_copy(stats_v, stats_hbm.at[pl.ds((c * NS + s) * 8, 8)])
        plsc.subcore_barrier()
        for k in range(ZPT):
            cid = k * NS + s

            @pl.when(cid < NZC)
            def _():
                pltpu.sync_copy(acc_sp.at[pl.ds(cid * ZR, ZR)],
                                acc_hbm.at[pl.ds(cN + cid * ZR, ZR)])

    run = pl.kernel(
        body,
        out_type=[
            jax.ShapeDtypeStruct((2 * E, H), jnp.float32),
            jax.ShapeDtypeStruct((2 * N, D), jnp.float32),
            jax.ShapeDtypeStruct((NC * NS * 8, D), jnp.float32),
        ],
        mesh=mesh,
        scratch_types=[
            pltpu.VMEM((C,), jnp.int32),
            pltpu.VMEM((C,), jnp.int32),
            pltpu.VMEM((C,), jnp.int32),
            pltpu.VMEM((C, D), jnp.float32),
            pltpu.VMEM((C, D), jnp.float32),
            pltpu.VMEM((C, H), jnp.float32),
            pltpu.VMEM((C, D), jnp.float32),
            pltpu.VMEM((8, D), jnp.float32),
            pltpu.VMEM((ZR, D), jnp.float32),
            pltpu.VMEM_SHARED((N, D), jnp.float32),
            pltpu.SemaphoreType.DMA,
            pltpu.SemaphoreType.DMA,
            pltpu.SemaphoreType.DMA,
        ],
    )
    return run(src, dst, srctab, edtab, ew)


def _edge_final_body(m_ref, ea_ref, st_ref, g_ref, b_ref, y_ref, *, E):
    H = m_ref.shape[2]
    s_lo = jnp.sum(st_ref[:NS * 8], axis=0, keepdims=True)
    s_hi = jnp.sum(st_ref[NS * 8:], axis=0, keepdims=True)
    mean = jnp.concatenate([s_lo[:, :H], s_hi[:, :H]], axis=1) / E
    em2 = jnp.concatenate([s_lo[:, H:], s_hi[:, H:]], axis=1) / E
    var = em2 - mean * mean
    inv = lax.rsqrt(var + 1e-5)
    m = jnp.concatenate([m_ref[0], m_ref[1]], axis=1)
    yn = g_ref[...] * (m - mean) * inv + b_ref[...]
    y_ref[...] = ea_ref[...] + yn * jax.nn.sigmoid(yn)


def _edge_final(edge_attr, m_split, stats, gamma_e, beta_e):
    E, D = edge_attr.shape
    H = D // 2
    BE = 8000
    return pl.pallas_call(
        functools.partial(_edge_final_body, E=E),
        grid=(E // BE,),
        in_specs=[
            pl.BlockSpec((2, BE, H), lambda i: (0, i, 0)),
            pl.BlockSpec((BE, D), lambda i: (i, 0)),
            pl.BlockSpec((NC * NS * 8, D), lambda i: (0, 0)),
            pl.BlockSpec((1, D), lambda i: (0, 0)),
            pl.BlockSpec((1, D), lambda i: (0, 0)),
        ],
        out_specs=pl.BlockSpec((BE, D), lambda i: (i, 0)),
        out_shape=jax.ShapeDtypeStruct((E, D), jnp.float32),
    )(m_split, edge_attr, stats, gamma_e.reshape(1, D), beta_e.reshape(1, D))


def _node_final_body(x_ref, xsu_ref, acc_ref, g_ref, b_ref, out_ref):
    D = x_ref.shape[1]
    H = D // 2
    a0 = acc_ref[0]
    a1 = acc_ref[1]
    h = jnp.concatenate([a0[:, :H] / (a0[:, H:] + 1e-6),
                         a1[:, :H] / (a1[:, H:] + 1e-6)], axis=1)
    xo = xsu_ref[...] + h
    mu = jnp.mean(xo, axis=0, keepdims=True)
    var = jnp.mean((xo - mu) * (xo - mu), axis=0, keepdims=True)
    xn = g_ref[...] * (xo - mu) * lax.rsqrt(var + 1e-5) + b_ref[...]
    out_ref[...] = x_ref[...] + xn * jax.nn.sigmoid(xn)


def _node_final(x, xsu, acc, gamma_n, beta_n):
    N, D = x.shape
    return pl.pallas_call(
        _node_final_body,
        out_shape=jax.ShapeDtypeStruct((N, D), jnp.float32),
    )(x, xsu, acc, gamma_n.reshape(1, D), beta_n.reshape(1, D))


def kernel(x, edge_index, edge_attr, W_src, b_src, W_dst, b_dst, W_eg, b_eg,
           W_su, b_su, W_du, b_du, gamma_n, beta_n, gamma_e, beta_e):
    N, D = x.shape
    E = edge_index.shape[1]
    H = D // 2
    src = edge_index[0]
    dst = edge_index[1]

    srctab, edtab, xsu = _node_linear(x, W_src, b_src, W_dst, b_dst,
                                      W_du, b_du, W_su, b_su)
    ew = _edge_linear(edge_attr, W_eg, b_eg)

    m_flat, acc_flat, stats = _sc_message(
        src, dst, srctab.reshape(2 * N, D), edtab,
        ew.reshape(2 * E, H), N, E, D)

    x_out = _node_final(x, xsu, acc_flat.reshape(2, N, D), gamma_n, beta_n)
    y_out = _edge_final(edge_attr, m_flat.reshape(2, E, H), stats,
                        gamma_e, beta_e)
    return (x_out, y_out)
